# baseline (speedup 1.0000x reference)
import functools

import jax
import jax.numpy as jnp
from jax.experimental import pallas as pl
from jax.experimental.pallas import tpu as pltpu

H = 16


def _proj_kernel(x_ref, wdq_ref, wuq_ref, wuq_r_ref, wdkv_ref, wuk_ref,
                 wuk_r_ref, wuv_ref, cos_ref, sin_ref,
                 q_ref, k_ref, v_ref):
    x = x_ref[...]
    cos = cos_ref[...]
    sin = sin_ref[...]
    q_lat = jnp.dot(x, wdq_ref[...], preferred_element_type=jnp.float32)
    qa = jnp.dot(q_lat, wuq_ref[...], preferred_element_type=jnp.float32)
    qb = jnp.dot(q_lat, wuq_r_ref[...], preferred_element_type=jnp.float32)
    q_ref[...] = (qa * cos + qb * sin) * 0.125
    kv = jnp.dot(x, wdkv_ref[...], preferred_element_type=jnp.float32)
    ka = jnp.dot(kv, wuk_ref[...], preferred_element_type=jnp.float32)
    kb = jnp.dot(kv, wuk_r_ref[...], preferred_element_type=jnp.float32)
    k_ref[...] = ka * cos + kb * sin
    v_ref[...] = jnp.dot(kv, wuv_ref[...], preferred_element_type=jnp.float32)


def _attn_kernel(q_ref, k_ref, v_ref, o_ref, *, tq, tk, hd):
    iq = pl.program_id(1)
    q = q_ref[0]

    def body(j, carry):
        m, l, acc = carry
        k_blk = k_ref[0, pl.ds(j * tk, tk), :]
        v_blk = v_ref[0, pl.ds(j * tk, tk), :]
        s = jax.lax.dot_general(q, k_blk, (((1,), (1,)), ((), ())),
                                preferred_element_type=jnp.float32)
        row = jax.lax.broadcasted_iota(jnp.int32, (tq, tk), 0) + iq * tq
        col = jax.lax.broadcasted_iota(jnp.int32, (tq, tk), 1) + j * tk
        s = jnp.where(col <= row, s, -1e30)
        m_new = jnp.maximum(m, jnp.max(s, axis=1, keepdims=True))
        p = jnp.exp(s - m_new)
        corr = jnp.exp(m - m_new)
        l = l * corr + jnp.sum(p, axis=1, keepdims=True)
        acc = acc * corr + jnp.dot(p, v_blk, preferred_element_type=jnp.float32)
        return m_new, l, acc

    m0 = jnp.full((tq, 1), -1e30, jnp.float32)
    l0 = jnp.zeros((tq, 1), jnp.float32)
    acc0 = jnp.zeros((tq, hd), jnp.float32)
    m, l, acc = jax.lax.fori_loop(0, iq + 1, body, (m0, l0, acc0))
    o_ref[0] = acc / l


def _moe1_kernel(ctx_ref, wo_ref, sw1_ref, sw2_ref, rw_ref,
                 t_ref, base_ref, gates_ref):
    ctx = ctx_ref[...]
    t = jnp.dot(ctx, wo_ref[...], preferred_element_type=jnp.float32)
    t_ref[...] = t
    sh0 = jnp.dot(jax.nn.silu(jnp.dot(t, sw1_ref[0], preferred_element_type=jnp.float32)),
                  sw2_ref[0], preferred_element_type=jnp.float32)
    sh1 = jnp.dot(jax.nn.silu(jnp.dot(t, sw1_ref[1], preferred_element_type=jnp.float32)),
                  sw2_ref[1], preferred_element_type=jnp.float32)
    base_ref[...] = t + sh0 + sh1

    logits = jnp.dot(t, rw_ref[...], preferred_element_type=jnp.float32)
    probs = jax.nn.softmax(logits, axis=-1)
    ncols = probs.shape[1]
    iota = jax.lax.broadcasted_iota(jnp.int32, probs.shape, 1)
    v1 = jnp.max(probs, axis=1, keepdims=True)
    i1 = jnp.min(jnp.where(probs == v1, iota, ncols), axis=1, keepdims=True)
    m1 = iota == i1
    p2 = jnp.where(m1, -jnp.inf, probs)
    v2 = jnp.max(p2, axis=1, keepdims=True)
    i2 = jnp.min(jnp.where(p2 == v2, iota, ncols), axis=1, keepdims=True)
    m2 = iota == i2
    denom = v1 + v2
    gates_ref[...] = jnp.where(m1, v1 / denom, 0.0) + jnp.where(m2, v2 / denom, 0.0)


def _routed_kernel(t_ref, base_ref, g_ref, w1_ref, w2_ref, o_ref):
    e = pl.program_id(1)
    t = t_ref[...]
    h = jax.nn.silu(jnp.dot(t, w1_ref[0], preferred_element_type=jnp.float32))
    y = jnp.dot(h, w2_ref[0], preferred_element_type=jnp.float32)
    g = g_ref[...]
    iota = jax.lax.broadcasted_iota(jnp.int32, g.shape, 1)
    g_col = jnp.sum(jnp.where(iota == e, g, 0.0), axis=1, keepdims=True)
    contrib = g_col * y

    @pl.when(e == 0)
    def _():
        o_ref[...] = base_ref[...] + contrib

    @pl.when(e != 0)
    def _():
        o_ref[...] += contrib


def _rope_tables(S, D, hd):
    half = hd // 2
    freqs = 1.0 / (10000.0 ** (jnp.arange(half, dtype=jnp.float32) / half))
    ang = jnp.arange(S, dtype=jnp.float32)[:, None] * freqs[None, :]
    cos = jnp.concatenate([jnp.cos(ang), jnp.cos(ang)], axis=1)
    sin = jnp.concatenate([jnp.sin(ang), jnp.sin(ang)], axis=1)
    reps = D // hd
    return jnp.tile(cos, (1, reps)), jnp.tile(sin, (1, reps))


def _rot_weight(w, hd):
    n, D = w.shape
    half = hd // 2
    w3 = w.reshape(n, D // hd, hd)
    return jnp.concatenate([-w3[..., half:], w3[..., :half]], axis=-1).reshape(n, D)


def kernel(x, Wdq, Wuq, Wdkv, Wuk, Wuv, Wo, router_w, shared_w1, shared_w2,
           routed_w1, routed_w2):
    B, S, D = x.shape
    hd = D // H
    n_lat = Wdq.shape[1]
    N_r = router_w.shape[1]
    dh = shared_w1.shape[2]
    TQ = 256
    nt = S // TQ

    x2 = x.reshape(S, D)
    cos, sin = _rope_tables(S, D, hd)
    Wuq_r = _rot_weight(Wuq, hd)
    Wuk_r = _rot_weight(Wuk, hd)

    full = lambda shape: pl.BlockSpec(shape, lambda i: (0,) * len(shape))
    row_tile = pl.BlockSpec((TQ, D), lambda i: (i, 0))
    q, k, v = pl.pallas_call(
        _proj_kernel,
        grid=(nt,),
        in_specs=[
            row_tile,
            full((D, n_lat)), full((n_lat, D)), full((n_lat, D)),
            full((D, n_lat)), full((n_lat, D)), full((n_lat, D)),
            full((n_lat, D)),
            row_tile, row_tile,
        ],
        out_specs=[row_tile, row_tile, row_tile],
        out_shape=[jax.ShapeDtypeStruct((S, D), jnp.float32)] * 3,
        compiler_params=pltpu.CompilerParams(
            dimension_semantics=("arbitrary",)),
    )(x2, Wdq, Wuq, Wuq_r, Wdkv, Wuk, Wuk_r, Wuv, cos, sin)

    def heads(t):
        return t.reshape(S, H, hd).transpose(1, 0, 2)
    qh, kh, vh = heads(q), heads(k), heads(v)
    ctx = pl.pallas_call(
        functools.partial(_attn_kernel, tq=TQ, tk=TQ, hd=hd),
        grid=(H, nt),
        in_specs=[
            pl.BlockSpec((1, TQ, hd), lambda h, i: (h, i, 0)),
            pl.BlockSpec((1, S, hd), lambda h, i: (h, 0, 0)),
            pl.BlockSpec((1, S, hd), lambda h, i: (h, 0, 0)),
        ],
        out_specs=pl.BlockSpec((1, TQ, hd), lambda h, i: (h, i, 0)),
        out_shape=jax.ShapeDtypeStruct((H, S, hd), jnp.float32),
        compiler_params=pltpu.CompilerParams(
            dimension_semantics=("arbitrary", "arbitrary")),
    )(qh, kh, vh)
    ctx2 = ctx.transpose(1, 0, 2).reshape(S, D)

    t_out, base, gates = pl.pallas_call(
        _moe1_kernel,
        grid=(nt,),
        in_specs=[
            row_tile,
            full((D, D)),
            full(shared_w1.shape), full(shared_w2.shape),
            full((D, N_r)),
        ],
        out_specs=[row_tile, row_tile,
                   pl.BlockSpec((TQ, N_r), lambda i: (i, 0))],
        out_shape=[
            jax.ShapeDtypeStruct((S, D), jnp.float32),
            jax.ShapeDtypeStruct((S, D), jnp.float32),
            jax.ShapeDtypeStruct((S, N_r), jnp.float32),
        ],
        compiler_params=pltpu.CompilerParams(
            dimension_semantics=("arbitrary",)),
    )(ctx2, Wo, shared_w1, shared_w2, router_w)

    out = pl.pallas_call(
        _routed_kernel,
        grid=(nt, N_r),
        in_specs=[
            pl.BlockSpec((TQ, D), lambda i, e: (i, 0)),
            pl.BlockSpec((TQ, D), lambda i, e: (i, 0)),
            pl.BlockSpec((TQ, N_r), lambda i, e: (i, 0)),
            pl.BlockSpec((1, D, dh), lambda i, e: (e, 0, 0)),
            pl.BlockSpec((1, dh, D), lambda i, e: (e, 0, 0)),
        ],
        out_specs=pl.BlockSpec((TQ, D), lambda i, e: (i, 0)),
        out_shape=jax.ShapeDtypeStruct((S, D), jnp.float32),
        compiler_params=pltpu.CompilerParams(
            dimension_semantics=("arbitrary", "arbitrary")),
    )(t_out, base, gates, routed_w1, routed_w2)

    return out.reshape(B, S, D)
